# baseline (speedup 1.0000x reference)
import jax
import jax.numpy as jnp
from jax.experimental import pallas as pl
from jax.experimental.pallas import tpu as pltpu

TBM = 256
TBP = TBM + 8
NBUF = 2


def _body(hbm_ref, w_ref, pi_ref, b_ref, out_ref, buf, sems):
    tc = pl.program_id(0)
    nst = pl.num_programs(0)
    nb = buf.shape[1]
    t_total = hbm_ref.shape[1]

    def start(step, slot):
        off = jnp.minimum(step * TBM, t_total - TBP)
        for bi in range(nb):
            pltpu.make_async_copy(
                hbm_ref.at[bi, pl.ds(off, TBP), :],
                buf.at[slot, bi],
                sems.at[slot, bi],
            ).start()

    def wait(slot):
        for bi in range(nb):
            pltpu.make_async_copy(
                hbm_ref.at[0, pl.ds(0, TBP), :],
                buf.at[slot, bi],
                sems.at[slot, bi],
            ).wait()

    @pl.when(tc == 0)
    def _():
        start(0, 0)

    @pl.when(tc + 1 < nst)
    def _():
        start(tc + 1, (tc + 1) % NBUF)

    slot = tc % NBUF
    wait(slot)
    w = w_ref[...]
    scale = pi_ref[...]
    bias = b_ref[...]
    for bi in range(nb):
        s = jax.lax.dot_general(
            buf[slot, bi], w, (((1,), (1,)), ((), ())),
            preferred_element_type=jnp.float32,
        )

        @pl.when(tc + 1 < nst)
        def _():
            out_ref[:, bi, :] = (s[1:TBM + 1] + bias) * scale

        @pl.when(tc + 1 == nst)
        def _():
            sl = jnp.concatenate([s[9:TBP], s[:1]], axis=0)
            out_ref[:, bi, :] = (sl + bias) * scale


def kernel(seq, pi, W, b):
    B, T, D = seq.shape
    E = W.shape[0]
    return pl.pallas_call(
        _body,
        grid=(T // TBM,),
        in_specs=[
            pl.BlockSpec(memory_space=pltpu.MemorySpace.HBM),
            pl.BlockSpec((E, D), lambda tc: (0, 0)),
            pl.BlockSpec((1, E), lambda tc: (0, 0)),
            pl.BlockSpec((1, E), lambda tc: (0, 0)),
        ],
        out_specs=pl.BlockSpec((TBM, B, E), lambda tc: (tc, 0, 0)),
        out_shape=jax.ShapeDtypeStruct((T - 1, B, E), jnp.float32),
        scratch_shapes=[
            pltpu.VMEM((NBUF, B, TBP, D), jnp.float32),
            pltpu.SemaphoreType.DMA((NBUF, B)),
        ],
        compiler_params=pltpu.CompilerParams(
            dimension_semantics=("arbitrary",),
        ),
    )(seq, W, pi.reshape(1, E), b.reshape(1, E))


# speedup vs baseline: 1.2581x; 1.2581x over previous
import jax
import jax.numpy as jnp
from jax.experimental import pallas as pl
from jax.experimental.pallas import tpu as pltpu

TB = 256


def _router_body(x_ref, w_ref, pi_ref, b_ref, out_ref):
    nb = x_ref.shape[0]
    w = w_ref[...]
    scale = pi_ref[...]
    bias = b_ref[...]
    for bi in range(nb):
        scores = jax.lax.dot_general(
            x_ref[bi], w, (((1,), (1,)), ((), ())),
            preferred_element_type=jnp.float32,
        )
        out_ref[:, bi, :] = (scores + bias) * scale


def kernel(seq, pi, W, b):
    B, T, D = seq.shape
    E = W.shape[0]
    grid = (T // TB,)
    full = pl.pallas_call(
        _router_body,
        grid=grid,
        in_specs=[
            pl.BlockSpec((B, TB, D), lambda tc: (0, tc, 0)),
            pl.BlockSpec((E, D), lambda tc: (0, 0)),
            pl.BlockSpec((1, E), lambda tc: (0, 0)),
            pl.BlockSpec((1, E), lambda tc: (0, 0)),
        ],
        out_specs=pl.BlockSpec((TB, B, E), lambda tc: (tc, 0, 0)),
        out_shape=jax.ShapeDtypeStruct((T, B, E), jnp.float32),
    )(seq, W, pi.reshape(1, E), b.reshape(1, E))
    return full[1:]
